# baseline (speedup 1.0000x reference)
import functools

import jax
import jax.numpy as jnp
from jax import lax
from jax.experimental import pallas as pl
from jax.experimental.pallas import tpu as pltpu
from jax.experimental.pallas import tpu_sc as plsc

BATCH = 4096
NSTEP = 11
LENGTH = 20
INPUT_SIZE = 64

NC = 2
NS = 16
NW = NC * NS
B_PER_W = BATCH // NW
GROUP = 16
NGROUP = B_PER_W // GROUP


def _gather_rows(weight, steps):
    mesh = plsc.VectorSubcoreMesh(core_axis_name="c", subcore_axis_name="s")

    @functools.partial(
        pl.kernel,
        mesh=mesh,
        out_type=jax.ShapeDtypeStruct((BATCH, LENGTH, INPUT_SIZE),
                                      jnp.float32),
        scratch_types=[
            pltpu.VMEM((B_PER_W,), jnp.int32),
            pltpu.VMEM((GROUP, LENGTH, INPUT_SIZE), jnp.float32),
            pltpu.SemaphoreType.DMA,
        ],
    )
    def k(weight_hbm, steps_hbm, out_hbm, steps_v, rows_v, sem):
        wid = lax.axis_index("s") * NC + lax.axis_index("c")
        start = wid * B_PER_W
        pltpu.sync_copy(steps_hbm.at[pl.ds(start, B_PER_W)], steps_v)

        @pl.loop(0, NGROUP)
        def _(g):
            base = g * GROUP
            svec = steps_v[pl.ds(base, GROUP)]
            copies = []
            for j in range(GROUP):
                copies.append(
                    pltpu.make_async_copy(
                        weight_hbm.at[start + base + j, svec[j]],
                        out_hbm.at[start + base + j], sem))
            for c in copies:
                c.start()
            for c in copies:
                c.wait()

    return k(weight, steps)


def kernel(embeddings, conditions, steps, weight, lengths):
    out = _gather_rows(weight, steps.astype(jnp.int32))
    return (out, lengths)


# speedup vs baseline: 4.3375x; 4.3375x over previous
import functools

import jax
import jax.numpy as jnp
from jax import lax
from jax.experimental import pallas as pl
from jax.experimental.pallas import tpu as pltpu
from jax.experimental.pallas import tpu_sc as plsc

BATCH = 4096
NSTEP = 11
LENGTH = 20
INPUT_SIZE = 64

NC = 2
NS = 16
NW = NC * NS
B_PER_W = BATCH // NW
GROUP = 16
NGROUP = B_PER_W // GROUP


def _gather_rows(weight, steps):
    mesh = plsc.VectorSubcoreMesh(core_axis_name="c", subcore_axis_name="s")

    @functools.partial(
        pl.kernel,
        mesh=mesh,
        out_type=jax.ShapeDtypeStruct((BATCH, LENGTH, INPUT_SIZE),
                                      jnp.float32),
        scratch_types=[
            pltpu.VMEM((B_PER_W,), jnp.int32),
            pltpu.VMEM((GROUP, LENGTH, INPUT_SIZE), jnp.float32),
            pltpu.VMEM((GROUP, LENGTH, INPUT_SIZE), jnp.float32),
            pltpu.SemaphoreType.DMA,
            pltpu.SemaphoreType.DMA,
        ],
    )
    def k(weight_hbm, steps_hbm, out_hbm, steps_v, rows_a, rows_b, sem_g,
          sem_w):
        wid = lax.axis_index("s") * NC + lax.axis_index("c")
        start = wid * B_PER_W
        pltpu.sync_copy(steps_hbm.at[pl.ds(start, B_PER_W)], steps_v)

        bufs = (rows_a, rows_b)

        def fire_gather(g):
            base = g * GROUP
            svec = steps_v[pl.ds(base, GROUP)]
            buf = bufs[g % 2]
            copies = []
            for j in range(GROUP):
                copies.append(
                    pltpu.make_async_copy(
                        weight_hbm.at[start + base + j, svec[j]],
                        buf.at[j], sem_g))
            for c in copies:
                c.start()
            return copies

        wb = [None] * NGROUP
        pending = fire_gather(0)
        for g in range(NGROUP):
            if g + 1 < NGROUP:
                if g - 1 >= 0:
                    wb[g - 1].wait()
                nxt = fire_gather(g + 1)
            for c in pending:
                c.wait()
            wb[g] = pltpu.make_async_copy(
                bufs[g % 2], out_hbm.at[pl.ds(start + g * GROUP, GROUP)],
                sem_w)
            wb[g].start()
            if g + 1 < NGROUP:
                pending = nxt
        wb[NGROUP - 2].wait()
        wb[NGROUP - 1].wait()

    return k(weight, steps)


def kernel(embeddings, conditions, steps, weight, lengths):
    out = _gather_rows(weight, steps.astype(jnp.int32))
    return (out, lengths)
